# Initial kernel scaffold; baseline (speedup 1.0000x reference)
import jax
import jax.numpy as jnp
from jax.experimental import pallas as pl


def kernel(in_embed, out_embed, input_labels, pos_labels, neg_labels, pairs):
    raise NotImplementedError("write your pallas kernel here")


# speedup vs baseline: 22.8942x; 22.8942x over previous
import functools

import jax
import jax.numpy as jnp
from jax import lax
from jax.experimental import pallas as pl
from jax.experimental.pallas import tpu as pltpu
from jax.experimental.pallas import tpu_sc as plsc

_VOCAB = 100000
_EMBED = 64
_B = 16384
_C = 20
_NEG = 100
_LE_LAMBDA = 1e-08
_NPAIR = 16
_CP = 32
_NEGP = 112

_NW = 32
_SPW = _B // _NW
_S = 8
_NCHUNK = _SPW // _S


def _sc_dots_body(in_embed, out_embed, il, plf, nlf, pidx,
                  pos_dot_hbm, neg_dot_hbm, pair_rows_hbm,
                  idx_in_v, idx_pos_v, idx_neg_v,
                  in_rows_v, pos_rows_v, neg_rows_v,
                  pos_dot_v, neg_dot_v, pair_idx_v, pair_rows_v, sem):
    cid = lax.axis_index("c")
    sid = lax.axis_index("s")
    wid = sid * 2 + cid

    @pl.when(wid == 0)
    def _():
        pltpu.sync_copy(pidx, pair_idx_v)
        pltpu.async_copy(in_embed.at[pair_idx_v], pair_rows_v, sem).wait()
        pltpu.sync_copy(pair_rows_v, pair_rows_hbm)

    def chunk_body(g, carry):
        base = wid * _SPW + g * _S
        pltpu.sync_copy(il.at[pl.ds(base, _S)], idx_in_v)
        pltpu.sync_copy(plf.at[pl.ds(base * _C, _S * _C)], idx_pos_v)
        pltpu.sync_copy(nlf.at[pl.ds(base * _NEG, _S * _NEG)], idx_neg_v)
        handles = [pltpu.async_copy(in_embed.at[idx_in_v], in_rows_v, sem)]
        handles.append(pltpu.async_copy(
            out_embed.at[idx_pos_v.at[pl.ds(0, 128)]],
            pos_rows_v.at[pl.ds(0, 128)], sem))
        handles.append(pltpu.async_copy(
            out_embed.at[idx_pos_v.at[pl.ds(128, 32)]],
            pos_rows_v.at[pl.ds(128, 32)], sem))
        for j in range(6):
            handles.append(pltpu.async_copy(
                out_embed.at[idx_neg_v.at[pl.ds(j * 128, 128)]],
                neg_rows_v.at[pl.ds(j * 128, 128)], sem))
        handles.append(pltpu.async_copy(
            out_embed.at[idx_neg_v.at[pl.ds(768, 32)]],
            neg_rows_v.at[pl.ds(768, 32)], sem))
        for h in handles:
            h.wait()

        lane = lax.iota(jnp.int32, 16)
        perms = [lane ^ k for k in (8, 4, 2, 1)]
        lmasks = [lane == l for l in range(16)]
        zero16 = jnp.zeros((16,), jnp.float32)

        def sample_body(i, carry2):
            u = [in_rows_v[i, pl.ds(16 * j, 16)] for j in range(4)]
            un = [-uj for uj in u]

            def dot_all_lanes(rows_v, r, uv):
                acc = rows_v[r, pl.ds(0, 16)] * uv[0]
                for j in range(1, 4):
                    acc = acc + rows_v[r, pl.ds(16 * j, 16)] * uv[j]
                for p in perms:
                    acc = acc + jnp.take_along_axis(acc, p, axis=0)
                return acc

            for g in range(_CP // 16):
                vec = zero16
                for l in range(16):
                    c = g * 16 + l
                    if c < _C:
                        s = dot_all_lanes(pos_rows_v, i * _C + c, u)
                        vec = jnp.where(lmasks[l], s, vec)
                pos_dot_v[i, pl.ds(g * 16, 16)] = vec
            for g in range(_NEGP // 16):
                vec = zero16
                for l in range(16):
                    n = g * 16 + l
                    if n < _NEG:
                        s = dot_all_lanes(neg_rows_v, i * _NEG + n, un)
                        vec = jnp.where(lmasks[l], s, vec)
                neg_dot_v[i, pl.ds(g * 16, 16)] = vec
            return carry2

        lax.fori_loop(0, _S, sample_body, None)
        pltpu.sync_copy(pos_dot_v, pos_dot_hbm.at[pl.ds(base, _S)])
        pltpu.sync_copy(neg_dot_v, neg_dot_hbm.at[pl.ds(base, _S)])
        return carry

    lax.fori_loop(0, _NCHUNK, chunk_body, None)


_sc_dots = pl.kernel(
    _sc_dots_body,
    out_type=[
        jax.ShapeDtypeStruct((_B, _CP), jnp.float32),
        jax.ShapeDtypeStruct((_B, _NEGP), jnp.float32),
        jax.ShapeDtypeStruct((2 * _NPAIR, _EMBED), jnp.float32),
    ],
    mesh=plsc.VectorSubcoreMesh(core_axis_name="c", subcore_axis_name="s"),
    compiler_params=pltpu.CompilerParams(use_tc_tiling_on_sc=False),
    scratch_types=[
        pltpu.VMEM((_S,), jnp.int32),
        pltpu.VMEM((_S * _C,), jnp.int32),
        pltpu.VMEM((_S * _NEG,), jnp.int32),
        pltpu.VMEM((_S, _EMBED), jnp.float32),
        pltpu.VMEM((_S * _C, _EMBED), jnp.float32),
        pltpu.VMEM((_S * _NEG, _EMBED), jnp.float32),
        pltpu.VMEM((_S, _CP), jnp.float32),
        pltpu.VMEM((_S, _NEGP), jnp.float32),
        pltpu.VMEM((2 * _NPAIR,), jnp.int32),
        pltpu.VMEM((2 * _NPAIR, _EMBED), jnp.float32),
        pltpu.SemaphoreType.DMA,
    ],
)


def _log_sigmoid(x):
    return jnp.minimum(x, 0.0) - jnp.log1p(jnp.exp(-jnp.abs(x)))


def _tc_reduce_body(pos_ref, neg_ref, pr_ref, loss_ref, hier_ref):
    i = pl.program_id(0)
    pmask = lax.broadcasted_iota(jnp.int32, (_TC_BLK, _CP), 1) < _C
    nmask = lax.broadcasted_iota(jnp.int32, (_TC_BLK, _NEGP), 1) < _NEG
    s = (jnp.sum(jnp.where(pmask, _log_sigmoid(pos_ref[...]), 0.0))
         + jnp.sum(jnp.where(nmask, _log_sigmoid(neg_ref[...]), 0.0)))
    part = -s / _B

    @pl.when(i == 0)
    def _():
        d = pr_ref[0:_NPAIR, :] - pr_ref[_NPAIR:2 * _NPAIR, :]
        h = 0.5 * _LE_LAMBDA * jnp.sum(d * d)
        hier_ref[0, 0] = h
        loss_ref[0, 0] = part + h

    @pl.when(i > 0)
    def _():
        loss_ref[0, 0] = loss_ref[0, 0] + part


_TC_BLK = 1024


def _tc_reduce(pos_dot, neg_dot, pair_rows):
    return pl.pallas_call(
        _tc_reduce_body,
        grid=(_B // _TC_BLK,),
        in_specs=[
            pl.BlockSpec((_TC_BLK, _CP), lambda i: (i, 0)),
            pl.BlockSpec((_TC_BLK, _NEGP), lambda i: (i, 0)),
            pl.BlockSpec((2 * _NPAIR, _EMBED), lambda i: (0, 0)),
        ],
        out_specs=[
            pl.BlockSpec((1, 1), lambda i: (0, 0), memory_space=pltpu.SMEM),
            pl.BlockSpec((1, 1), lambda i: (0, 0), memory_space=pltpu.SMEM),
        ],
        out_shape=[
            jax.ShapeDtypeStruct((1, 1), jnp.float32),
            jax.ShapeDtypeStruct((1, 1), jnp.float32),
        ],
    )(pos_dot, neg_dot, pair_rows)


def kernel(in_embed, out_embed, input_labels, pos_labels, neg_labels, pairs):
    il = input_labels.astype(jnp.int32)
    plf = pos_labels.reshape(-1).astype(jnp.int32)
    nlf = neg_labels.reshape(-1).astype(jnp.int32)
    pidx = jnp.concatenate([pairs[:, 0], pairs[:, 1]]).astype(jnp.int32)
    pos_dot, neg_dot, pair_rows = _sc_dots(
        in_embed, out_embed, il, plf, nlf, pidx)
    loss, hier = _tc_reduce(pos_dot, neg_dot, pair_rows)
    return (loss[0, 0], hier[0, 0])


# speedup vs baseline: 36.2585x; 1.5837x over previous
import jax
import jax.numpy as jnp
from jax import lax
from jax.experimental import pallas as pl
from jax.experimental.pallas import tpu as pltpu
from jax.experimental.pallas import tpu_sc as plsc

_VOCAB = 100000
_EMBED = 64
_B = 16384
_C = 20
_NEG = 100
_LE_LAMBDA = 1e-08
_NPAIR = 16
_CP = 32
_NEGP = 112

_NW = 32
_SPW = _B // _NW
_S = 8
_NCHUNK = _SPW // _S


def _sc_dots_body(in_embed, out_embed, il, plf, nlf, pidx,
                  pos_dot_hbm, neg_dot_hbm, pair_rows_hbm,
                  *scr):
    idx_in = scr[0:2]
    idx_pos = scr[2:4]
    idx_neg = scr[4:6]
    in_rows = scr[6:8]
    pos_rows = scr[8:10]
    neg_rows = scr[10:12]
    pos_dot = scr[12:14]
    neg_dot = scr[14:16]
    pair_idx_v, pair_rows_v, gsem, isem, osem = scr[16:21]

    cid = lax.axis_index("c")
    sid = lax.axis_index("s")
    wid = sid * 2 + cid

    @pl.when(wid == 0)
    def _():
        pltpu.sync_copy(pidx, pair_idx_v)
        pltpu.async_copy(in_embed.at[pair_idx_v], pair_rows_v, gsem).wait()
        pltpu.sync_copy(pair_rows_v, pair_rows_hbm)

    def issue_idx(g, b):
        base = wid * _SPW + g * _S
        pltpu.async_copy(il.at[pl.ds(base, _S)], idx_in[b], isem)
        pltpu.async_copy(plf.at[pl.ds(base * _C, _S * _C)], idx_pos[b], isem)
        pltpu.async_copy(nlf.at[pl.ds(base * _NEG, _S * _NEG)], idx_neg[b], isem)

    def wait_idx(b):
        pltpu.make_async_copy(il.at[pl.ds(0, _S)], idx_in[b], isem).wait()
        pltpu.make_async_copy(plf.at[pl.ds(0, _S * _C)], idx_pos[b], isem).wait()
        pltpu.make_async_copy(nlf.at[pl.ds(0, _S * _NEG)], idx_neg[b], isem).wait()

    def issue_gathers(b):
        pltpu.async_copy(in_embed.at[idx_in[b]], in_rows[b], gsem)
        pltpu.async_copy(out_embed.at[idx_pos[b].at[pl.ds(0, 128)]],
                         pos_rows[b].at[pl.ds(0, 128)], gsem)
        pltpu.async_copy(out_embed.at[idx_pos[b].at[pl.ds(128, 32)]],
                         pos_rows[b].at[pl.ds(128, 32)], gsem)
        for j in range(6):
            pltpu.async_copy(out_embed.at[idx_neg[b].at[pl.ds(j * 128, 128)]],
                             neg_rows[b].at[pl.ds(j * 128, 128)], gsem)
        pltpu.async_copy(out_embed.at[idx_neg[b].at[pl.ds(768, 32)]],
                         neg_rows[b].at[pl.ds(768, 32)], gsem)

    def wait_gathers(b):
        pltpu.make_async_copy(in_embed.at[idx_in[b]], in_rows[b], gsem).wait()
        pltpu.make_async_copy(out_embed.at[idx_pos[b]], pos_rows[b], gsem).wait()
        pltpu.make_async_copy(out_embed.at[idx_neg[b]], neg_rows[b], gsem).wait()

    def issue_out(g, b):
        base = wid * _SPW + g * _S
        pltpu.async_copy(pos_dot[b], pos_dot_hbm.at[pl.ds(base, _S)], osem)
        pltpu.async_copy(neg_dot[b], neg_dot_hbm.at[pl.ds(base, _S)], osem)

    def wait_out(b):
        pltpu.make_async_copy(pos_dot[b], pos_dot_hbm.at[pl.ds(0, _S)], osem).wait()
        pltpu.make_async_copy(neg_dot[b], neg_dot_hbm.at[pl.ds(0, _S)], osem).wait()

    lane = lax.iota(jnp.int32, 16)
    perms = [lane ^ k for k in (8, 4, 2, 1)]
    lmasks = [lane == l for l in range(16)]
    zero16 = jnp.zeros((16,), jnp.float32)

    def compute(b):
        def sample_body(i, carry2):
            u = [in_rows[b][i, pl.ds(16 * j, 16)] for j in range(4)]
            un = [-uj for uj in u]

            def dot_all_lanes(rows_v, r, uv):
                acc = rows_v[r, pl.ds(0, 16)] * uv[0]
                for j in range(1, 4):
                    acc = acc + rows_v[r, pl.ds(16 * j, 16)] * uv[j]
                for p in perms:
                    acc = acc + jnp.take_along_axis(acc, p, axis=0)
                return acc

            for g16 in range(_CP // 16):
                vec = zero16
                for l in range(16):
                    c = g16 * 16 + l
                    if c < _C:
                        s = dot_all_lanes(pos_rows[b], i * _C + c, u)
                        vec = jnp.where(lmasks[l], s, vec)
                pos_dot[b][i, pl.ds(g16 * 16, 16)] = vec
            for g16 in range(_NEGP // 16):
                vec = zero16
                for l in range(16):
                    n = g16 * 16 + l
                    if n < _NEG:
                        s = dot_all_lanes(neg_rows[b], i * _NEG + n, un)
                        vec = jnp.where(lmasks[l], s, vec)
                neg_dot[b][i, pl.ds(g16 * 16, 16)] = vec
            return carry2

        lax.fori_loop(0, _S, sample_body, None)

    issue_idx(0, 0)
    wait_idx(0)
    issue_gathers(0)
    issue_idx(1, 1)

    def step_body(step, carry):
        for b in (0, 1):
            g = 2 * step + b
            g2 = jnp.minimum(g + 2, _NCHUNK - 1)
            wait_gathers(b)
            wait_idx(1 - b)
            issue_gathers(1 - b)
            issue_idx(g2, b)

            @pl.when(step >= 1)
            def _():
                wait_out(b)

            compute(b)
            issue_out(g, b)
        return carry

    lax.fori_loop(0, _NCHUNK // 2, step_body, None)

    wait_gathers(0)
    wait_idx(1)
    wait_out(0)
    wait_out(1)


_sc_dots = pl.kernel(
    _sc_dots_body,
    out_type=[
        jax.ShapeDtypeStruct((_B, _CP), jnp.float32),
        jax.ShapeDtypeStruct((_B, _NEGP), jnp.float32),
        jax.ShapeDtypeStruct((2 * _NPAIR, _EMBED), jnp.float32),
    ],
    mesh=plsc.VectorSubcoreMesh(core_axis_name="c", subcore_axis_name="s"),
    compiler_params=pltpu.CompilerParams(use_tc_tiling_on_sc=False),
    scratch_types=(
        [pltpu.VMEM((_S,), jnp.int32)] * 2
        + [pltpu.VMEM((_S * _C,), jnp.int32)] * 2
        + [pltpu.VMEM((_S * _NEG,), jnp.int32)] * 2
        + [pltpu.VMEM((_S, _EMBED), jnp.float32)] * 2
        + [pltpu.VMEM((_S * _C, _EMBED), jnp.float32)] * 2
        + [pltpu.VMEM((_S * _NEG, _EMBED), jnp.float32)] * 2
        + [pltpu.VMEM((_S, _CP), jnp.float32)] * 2
        + [pltpu.VMEM((_S, _NEGP), jnp.float32)] * 2
        + [
            pltpu.VMEM((2 * _NPAIR,), jnp.int32),
            pltpu.VMEM((2 * _NPAIR, _EMBED), jnp.float32),
            pltpu.SemaphoreType.DMA,
            pltpu.SemaphoreType.DMA,
            pltpu.SemaphoreType.DMA,
        ]
    ),
)


def _log_sigmoid(x):
    return jnp.minimum(x, 0.0) - jnp.log1p(jnp.exp(-jnp.abs(x)))


def _tc_reduce_body(pos_ref, neg_ref, pr_ref, loss_ref, hier_ref):
    i = pl.program_id(0)
    pmask = lax.broadcasted_iota(jnp.int32, (_TC_BLK, _CP), 1) < _C
    nmask = lax.broadcasted_iota(jnp.int32, (_TC_BLK, _NEGP), 1) < _NEG
    s = (jnp.sum(jnp.where(pmask, _log_sigmoid(pos_ref[...]), 0.0))
         + jnp.sum(jnp.where(nmask, _log_sigmoid(neg_ref[...]), 0.0)))
    part = -s / _B

    @pl.when(i == 0)
    def _():
        d = pr_ref[0:_NPAIR, :] - pr_ref[_NPAIR:2 * _NPAIR, :]
        h = 0.5 * _LE_LAMBDA * jnp.sum(d * d)
        hier_ref[0, 0] = h
        loss_ref[0, 0] = part + h

    @pl.when(i > 0)
    def _():
        loss_ref[0, 0] = loss_ref[0, 0] + part


_TC_BLK = 1024


def _tc_reduce(pos_dot, neg_dot, pair_rows):
    return pl.pallas_call(
        _tc_reduce_body,
        grid=(_B // _TC_BLK,),
        in_specs=[
            pl.BlockSpec((_TC_BLK, _CP), lambda i: (i, 0)),
            pl.BlockSpec((_TC_BLK, _NEGP), lambda i: (i, 0)),
            pl.BlockSpec((2 * _NPAIR, _EMBED), lambda i: (0, 0)),
        ],
        out_specs=[
            pl.BlockSpec((1, 1), lambda i: (0, 0), memory_space=pltpu.SMEM),
            pl.BlockSpec((1, 1), lambda i: (0, 0), memory_space=pltpu.SMEM),
        ],
        out_shape=[
            jax.ShapeDtypeStruct((1, 1), jnp.float32),
            jax.ShapeDtypeStruct((1, 1), jnp.float32),
        ],
    )(pos_dot, neg_dot, pair_rows)


def kernel(in_embed, out_embed, input_labels, pos_labels, neg_labels, pairs):
    il = input_labels.astype(jnp.int32)
    plf = pos_labels.reshape(-1).astype(jnp.int32)
    nlf = neg_labels.reshape(-1).astype(jnp.int32)
    pidx = jnp.concatenate([pairs[:, 0], pairs[:, 1]]).astype(jnp.int32)
    pos_dot, neg_dot, pair_rows = _sc_dots(
        in_embed, out_embed, il, plf, nlf, pidx)
    loss, hier = _tc_reduce(pos_dot, neg_dot, pair_rows)
    return (loss[0, 0], hier[0, 0])


# speedup vs baseline: 37.7715x; 1.0417x over previous
import jax
import jax.numpy as jnp
from jax import lax
from jax.experimental import pallas as pl
from jax.experimental.pallas import tpu as pltpu
from jax.experimental.pallas import tpu_sc as plsc

_VOCAB = 100000
_EMBED = 64
_B = 16384
_C = 20
_NEG = 100
_LE_LAMBDA = 1e-08
_NPAIR = 16
_CP = 32
_NEGP = 112

_NW = 32
_SPW = _B // _NW
_S = 8
_NCHUNK = _SPW // _S


def _sc_dots_body(in_embed, out_embed, il, plf, nlf, pidx,
                  pos_dot_hbm, neg_dot_hbm, pair_rows_hbm,
                  *scr):
    idx_in = scr[0:2]
    idx_pos = scr[2:4]
    idx_neg = scr[4:6]
    in_rows = scr[6:8]
    pos_rows = scr[8:10]
    neg_rows = scr[10:12]
    pos_dot = scr[12:14]
    neg_dot = scr[14:16]
    pair_idx_v, pair_rows_v, gsem, isem, osem = scr[16:21]

    cid = lax.axis_index("c")
    sid = lax.axis_index("s")
    wid = sid * 2 + cid

    @pl.when(wid == 0)
    def _():
        pltpu.sync_copy(pidx, pair_idx_v)
        pltpu.async_copy(in_embed.at[pair_idx_v], pair_rows_v, gsem).wait()
        pltpu.sync_copy(pair_rows_v, pair_rows_hbm)

    def issue_idx(g, b):
        base = wid * _SPW + g * _S
        pltpu.async_copy(il.at[pl.ds(base, _S)], idx_in[b], isem)
        pltpu.async_copy(plf.at[pl.ds(base * _C, _S * _C)], idx_pos[b], isem)
        pltpu.async_copy(nlf.at[pl.ds(base * _NEG, _S * _NEG)], idx_neg[b], isem)

    def wait_idx(b):
        pltpu.make_async_copy(il.at[pl.ds(0, _S)], idx_in[b], isem).wait()
        pltpu.make_async_copy(plf.at[pl.ds(0, _S * _C)], idx_pos[b], isem).wait()
        pltpu.make_async_copy(nlf.at[pl.ds(0, _S * _NEG)], idx_neg[b], isem).wait()

    def issue_gathers(b):
        pltpu.async_copy(in_embed.at[idx_in[b]], in_rows[b], gsem)
        pltpu.async_copy(out_embed.at[idx_pos[b].at[pl.ds(0, 128)]],
                         pos_rows[b].at[pl.ds(0, 128)], gsem)
        pltpu.async_copy(out_embed.at[idx_pos[b].at[pl.ds(128, 32)]],
                         pos_rows[b].at[pl.ds(128, 32)], gsem)
        for j in range(6):
            pltpu.async_copy(out_embed.at[idx_neg[b].at[pl.ds(j * 128, 128)]],
                             neg_rows[b].at[pl.ds(j * 128, 128)], gsem)
        pltpu.async_copy(out_embed.at[idx_neg[b].at[pl.ds(768, 32)]],
                         neg_rows[b].at[pl.ds(768, 32)], gsem)

    def wait_gathers(b):
        pltpu.make_async_copy(in_embed.at[idx_in[b]], in_rows[b], gsem).wait()
        pltpu.make_async_copy(out_embed.at[idx_pos[b]], pos_rows[b], gsem).wait()
        pltpu.make_async_copy(out_embed.at[idx_neg[b]], neg_rows[b], gsem).wait()

    def issue_out(g, b):
        base = wid * _SPW + g * _S
        pltpu.async_copy(pos_dot[b], pos_dot_hbm.at[pl.ds(base, _S)], osem)
        pltpu.async_copy(neg_dot[b], neg_dot_hbm.at[pl.ds(base, _S)], osem)

    def wait_out(b):
        pltpu.make_async_copy(pos_dot[b], pos_dot_hbm.at[pl.ds(0, _S)], osem).wait()
        pltpu.make_async_copy(neg_dot[b], neg_dot_hbm.at[pl.ds(0, _S)], osem).wait()

    lane = lax.iota(jnp.int32, 16)
    perms = {k: lane ^ k for k in (1, 2, 4, 8)}
    kmasks = {k: (lane & k) == 0 for k in (1, 2, 4, 8)}

    def compute(b):
        def bfly(a, k):
            return a + jnp.take_along_axis(a, perms[k], axis=0)

        def merge(a, bv, k):
            return jnp.where(kmasks[k], bfly(a, k), bfly(bv, k))

        def sample_body(i, carry2):
            u = [in_rows[b][i, pl.ds(16 * j, 16)] for j in range(4)]
            un = [-uj for uj in u]

            def dot_partial(rows_v, r, uv):
                acc = rows_v[r, pl.ds(0, 16)] * uv[0]
                for j in range(1, 4):
                    acc = acc + rows_v[r, pl.ds(16 * j, 16)] * uv[j]
                return acc

            def group16(rows_v, r0, uv, count):
                level = [dot_partial(rows_v, r0 + j, uv) for j in range(count)]
                for k in (1, 2, 4, 8):
                    if len(level) == 1:
                        level = [bfly(level[0], k)]
                    else:
                        level = [merge(level[2 * m], level[2 * m + 1], k)
                                 for m in range(len(level) // 2)]
                return level[0]

            pos_dot[b][i, pl.ds(0, 16)] = group16(pos_rows[b], i * _C, u, 16)
            pos_dot[b][i, pl.ds(16, 16)] = group16(pos_rows[b], i * _C + 16, u, 4)
            for g16 in range(6):
                neg_dot[b][i, pl.ds(g16 * 16, 16)] = group16(
                    neg_rows[b], i * _NEG + g16 * 16, un, 16)
            neg_dot[b][i, pl.ds(96, 16)] = group16(
                neg_rows[b], i * _NEG + 96, un, 4)
            return carry2

        lax.fori_loop(0, _S, sample_body, None)

    issue_idx(0, 0)
    wait_idx(0)
    issue_gathers(0)
    issue_idx(1, 1)

    def step_body(step, carry):
        for b in (0, 1):
            g = 2 * step + b
            g2 = jnp.minimum(g + 2, _NCHUNK - 1)
            wait_gathers(b)
            wait_idx(1 - b)
            issue_gathers(1 - b)
            issue_idx(g2, b)

            @pl.when(step >= 1)
            def _():
                wait_out(b)

            compute(b)
            issue_out(g, b)
        return carry

    lax.fori_loop(0, _NCHUNK // 2, step_body, None)

    wait_gathers(0)
    wait_idx(1)
    wait_out(0)
    wait_out(1)


_sc_dots = pl.kernel(
    _sc_dots_body,
    out_type=[
        jax.ShapeDtypeStruct((_B, _CP), jnp.float32),
        jax.ShapeDtypeStruct((_B, _NEGP), jnp.float32),
        jax.ShapeDtypeStruct((2 * _NPAIR, _EMBED), jnp.float32),
    ],
    mesh=plsc.VectorSubcoreMesh(core_axis_name="c", subcore_axis_name="s"),
    compiler_params=pltpu.CompilerParams(use_tc_tiling_on_sc=False),
    scratch_types=(
        [pltpu.VMEM((_S,), jnp.int32)] * 2
        + [pltpu.VMEM((_S * _C,), jnp.int32)] * 2
        + [pltpu.VMEM((_S * _NEG,), jnp.int32)] * 2
        + [pltpu.VMEM((_S, _EMBED), jnp.float32)] * 2
        + [pltpu.VMEM((_S * _C, _EMBED), jnp.float32)] * 2
        + [pltpu.VMEM((_S * _NEG, _EMBED), jnp.float32)] * 2
        + [pltpu.VMEM((_S, _CP), jnp.float32)] * 2
        + [pltpu.VMEM((_S, _NEGP), jnp.float32)] * 2
        + [
            pltpu.VMEM((2 * _NPAIR,), jnp.int32),
            pltpu.VMEM((2 * _NPAIR, _EMBED), jnp.float32),
            pltpu.SemaphoreType.DMA,
            pltpu.SemaphoreType.DMA,
            pltpu.SemaphoreType.DMA,
        ]
    ),
)


def _log_sigmoid(x):
    return jnp.minimum(x, 0.0) - jnp.log1p(jnp.exp(-jnp.abs(x)))


def _tc_reduce_body(pos_ref, neg_ref, pr_ref, loss_ref, hier_ref):
    i = pl.program_id(0)
    pmask = lax.broadcasted_iota(jnp.int32, (_TC_BLK, _CP), 1) < _C
    nmask = lax.broadcasted_iota(jnp.int32, (_TC_BLK, _NEGP), 1) < _NEG
    s = (jnp.sum(jnp.where(pmask, _log_sigmoid(pos_ref[...]), 0.0))
         + jnp.sum(jnp.where(nmask, _log_sigmoid(neg_ref[...]), 0.0)))
    part = -s / _B

    @pl.when(i == 0)
    def _():
        d = pr_ref[0:_NPAIR, :] - pr_ref[_NPAIR:2 * _NPAIR, :]
        h = 0.5 * _LE_LAMBDA * jnp.sum(d * d)
        hier_ref[0, 0] = h
        loss_ref[0, 0] = part + h

    @pl.when(i > 0)
    def _():
        loss_ref[0, 0] = loss_ref[0, 0] + part


_TC_BLK = 1024


def _tc_reduce(pos_dot, neg_dot, pair_rows):
    return pl.pallas_call(
        _tc_reduce_body,
        grid=(_B // _TC_BLK,),
        in_specs=[
            pl.BlockSpec((_TC_BLK, _CP), lambda i: (i, 0)),
            pl.BlockSpec((_TC_BLK, _NEGP), lambda i: (i, 0)),
            pl.BlockSpec((2 * _NPAIR, _EMBED), lambda i: (0, 0)),
        ],
        out_specs=[
            pl.BlockSpec((1, 1), lambda i: (0, 0), memory_space=pltpu.SMEM),
            pl.BlockSpec((1, 1), lambda i: (0, 0), memory_space=pltpu.SMEM),
        ],
        out_shape=[
            jax.ShapeDtypeStruct((1, 1), jnp.float32),
            jax.ShapeDtypeStruct((1, 1), jnp.float32),
        ],
    )(pos_dot, neg_dot, pair_rows)


def kernel(in_embed, out_embed, input_labels, pos_labels, neg_labels, pairs):
    il = input_labels.astype(jnp.int32)
    plf = pos_labels.reshape(-1).astype(jnp.int32)
    nlf = neg_labels.reshape(-1).astype(jnp.int32)
    pidx = jnp.concatenate([pairs[:, 0], pairs[:, 1]]).astype(jnp.int32)
    pos_dot, neg_dot, pair_rows = _sc_dots(
        in_embed, out_embed, il, plf, nlf, pidx)
    loss, hier = _tc_reduce(pos_dot, neg_dot, pair_rows)
    return (loss[0, 0], hier[0, 0])
